# Initial kernel scaffold; baseline (speedup 1.0000x reference)
import jax
import jax.numpy as jnp
from jax.experimental import pallas as pl


def kernel(x, edge_index, edge_weight, W1, b1, W2, b2):
    raise NotImplementedError("write your pallas kernel here")


# speedup vs baseline: 17.5015x; 17.5015x over previous
import functools

import jax
import jax.numpy as jnp
from jax import lax
from jax.experimental import pallas as pl
from jax.experimental.pallas import tpu as pltpu
from jax.experimental.pallas import tpu_sc as plsc

N = 10000
NP = 10112
D = 128
NC = 2
NS = 16
NW = NC * NS
CH = 80
IDX_BITS = 14
ROWS_PER_TILE = NP // NS

_mesh = plsc.VectorSubcoreMesh(core_axis_name="c", subcore_axis_name="s")


def _zero_vmem(buf, rows, width):
  z = jnp.zeros((16,), jnp.float32)
  for i in range(rows):
    r = buf.at[i]
    for j in range(width // 16):
      r[pl.ds(j * 16, 16)] = z


def _zero_stripe(zbuf, zrows, shared, base_row):
  nfull = ROWS_PER_TILE // zrows
  for k in range(nfull):
    pltpu.sync_copy(zbuf, shared.at[pl.ds(base_row + k * zrows, zrows)])
  rem = ROWS_PER_TILE - nfull * zrows
  if rem:
    pltpu.sync_copy(zbuf.at[pl.ds(0, rem)],
                    shared.at[pl.ds(base_row + nfull * zrows, rem)])


def _copy_out_stripe(shared, out_core, base_row):
  for k in range(4):
    pltpu.sync_copy(shared.at[pl.ds(base_row + k * CH, CH)],
                    out_core.at[pl.ds(base_row + k * CH, CH)])
  r = ROWS_PER_TILE - 4 * CH
  pltpu.sync_copy(shared.at[pl.ds(base_row + 4 * CH, r)],
                  out_core.at[pl.ds(base_row + 4 * CH, r)])


def _make_deg_kernel(ept):

  @functools.partial(
      pl.kernel,
      out_type=jax.ShapeDtypeStruct((NW, NP), jnp.float32),
      mesh=_mesh,
      compiler_params=pltpu.CompilerParams(needs_layout_passes=False),
      scratch_types=[
          pltpu.VMEM((ept,), jnp.int32),
          pltpu.VMEM((ept,), jnp.float32),
          pltpu.VMEM((NP,), jnp.float32),
      ],
  )
  def _deg(pk_hbm, ew_hbm, out_hbm, pk_v, ew_v, tab_v):
    c = lax.axis_index("c")
    s = lax.axis_index("s")
    wid = c * NS + s
    z = jnp.zeros((16,), jnp.float32)

    def zbody(i, carry):
      tab_v[pl.ds(i * 16, 16)] = z
      return carry

    lax.fori_loop(0, NP // 16, zbody, 0)
    pltpu.sync_copy(pk_hbm.at[pl.ds(wid * ept, ept)], pk_v)
    pltpu.sync_copy(ew_hbm.at[pl.ds(wid * ept, ept)], ew_v)

    def body(i, carry):
      col = lax.shift_right_logical(pk_v[pl.ds(i * 16, 16)], IDX_BITS)
      plsc.addupdate_scatter(tab_v, [col], ew_v[pl.ds(i * 16, 16)])
      return carry

    lax.fori_loop(0, ept // 16, body, 0)
    pltpu.sync_copy(tab_v, out_hbm.at[wid])

  return _deg


def _make_agg_kernel(ept):
  nch = ept // CH
  assert ept % CH == 0 and nch % 2 == 0
  niter = nch // 2
  mask = (1 << IDX_BITS) - 1

  @functools.partial(
      pl.kernel,
      out_type=jax.ShapeDtypeStruct((NC, NP, D), jnp.float32),
      mesh=_mesh,
      scratch_types=[
          pltpu.VMEM((ept,), jnp.int32),
          pltpu.VMEM((ept,), jnp.float32),
          pltpu.VMEM((CH,), jnp.int32),
          pltpu.VMEM((CH,), jnp.int32),
          pltpu.VMEM((CH,), jnp.int32),
          pltpu.VMEM((CH,), jnp.int32),
          pltpu.VMEM((CH, D), jnp.float32),
          pltpu.VMEM((CH, D), jnp.float32),
          pltpu.SemaphoreType.DMA,
          pltpu.SemaphoreType.DMA,
          pltpu.VMEM_SHARED((NP, D), jnp.float32),
      ],
  )
  def _agg(hp_hbm, pk_hbm, ew_hbm, out_hbm, pk_v, ew_v, ridx_a, ridx_b,
           cidx_a, cidx_b, buf_a, buf_b, sem_a, sem_b, acc_sh):
    c = lax.axis_index("c")
    s = lax.axis_index("s")
    wid = c * NS + s
    _zero_vmem(buf_a, CH, D)
    base_row = s * ROWS_PER_TILE
    _zero_stripe(buf_a, CH, acc_sh, base_row)

    pltpu.sync_copy(pk_hbm.at[pl.ds(wid * ept, ept)], pk_v)
    pltpu.sync_copy(ew_hbm.at[pl.ds(wid * ept, ept)], ew_v)
    plsc.subcore_barrier()

    dn = lax.GatherDimensionNumbers(
        offset_dims=(), collapsed_slice_dims=(0,), start_index_map=(0,))
    lane_idx = [jnp.full((16, 1), l, jnp.int32) + jnp.zeros((16, 1), jnp.int32)
                for l in range(16)]

    def unpack(g, ridx, cidx):
      for t in range(CH // 16):
        v = pk_v[pl.ds(g * CH + t * 16, 16)]
        ridx[pl.ds(t * 16, 16)] = v & mask
        cidx[pl.ds(t * 16, 16)] = lax.shift_right_logical(v, IDX_BITS)

    def scale_scatter(g, buf, cidx):
      for t in range(CH // 16):
        ewg = ew_v[pl.ds(g * CH + t * 16, 16)]
        for l in range(16):
          w = lax.gather(ewg, lane_idx[l], dn, slice_sizes=(1,),
                         mode=lax.GatherScatterMode.PROMISE_IN_BOUNDS)
          rv = buf.at[t * 16 + l]
          for j in range(D // 16):
            rv[pl.ds(j * 16, 16)] = rv[pl.ds(j * 16, 16)] * w
      pltpu.sync_copy(buf, acc_sh.at[cidx], add=True)

    unpack(0, ridx_a, cidx_a)
    pltpu.async_copy(hp_hbm.at[ridx_a], buf_a, sem_a)

    def body(k, carry):
      g = 2 * k
      unpack(g + 1, ridx_b, cidx_b)
      pltpu.async_copy(hp_hbm.at[ridx_b], buf_b, sem_b)
      pltpu.make_async_copy(hp_hbm.at[ridx_a], buf_a, sem_a).wait()
      scale_scatter(g, buf_a, cidx_a)

      @pl.when(k < niter - 1)
      def _():
        unpack(g + 2, ridx_a, cidx_a)
        pltpu.async_copy(hp_hbm.at[ridx_a], buf_a, sem_a)

      pltpu.make_async_copy(hp_hbm.at[ridx_b], buf_b, sem_b).wait()
      scale_scatter(g + 1, buf_b, cidx_b)
      return carry

    lax.fori_loop(0, niter, body, 0)
    plsc.subcore_barrier()
    _copy_out_stripe(acc_sh, out_hbm.at[c], base_row)

  return _agg


def _tc1(x_ref, w1_ref, degp_ref, h1p_ref, dis_ref):
  ones = jnp.ones((NW, D), jnp.float32)
  deg2d = lax.dot_general(degp_ref[...], ones, (((0,), (0,)), ((), ())),
                          preferred_element_type=jnp.float32)
  dis = lax.rsqrt(deg2d[:N] + 1.0)
  h = jnp.dot(x_ref[...], w1_ref[...], preferred_element_type=jnp.float32)
  h1p_ref[...] = h * dis
  dis_ref[...] = dis


def _tc2(ap_ref, hp_ref, dis_ref, b_ref, w2_ref, h2p_ref):
  dis = dis_ref[...]
  z = (ap_ref[0, :N] + ap_ref[1, :N] + hp_ref[...]) * dis + b_ref[...]
  y = jnp.where(z > 0, z, 0.01 * z)
  h2 = jnp.dot(y, w2_ref[...], preferred_element_type=jnp.float32)
  h2p_ref[...] = h2 * dis


def _tc3(ap_ref, hp_ref, dis_ref, b_ref, out_ref):
  out_ref[...] = (ap_ref[0, :N] + ap_ref[1, :N] + hp_ref[...]) \
      * dis_ref[...] + b_ref[...]


@jax.jit
def kernel(x, edge_index, edge_weight, W1, b1, W2, b2):
  e = edge_index.shape[1]
  ept = -(-e // (NW * 2 * CH)) * 2 * CH
  ep = NW * ept
  row = edge_index[0].astype(jnp.int32)
  col = edge_index[1].astype(jnp.int32)
  packed = jnp.pad(row | (col << IDX_BITS), (0, ep - e))
  ew = jnp.pad(edge_weight, (0, ep - e))

  degp = _make_deg_kernel(ept)(packed, ew)

  h1p, dis = pl.pallas_call(
      _tc1,
      out_shape=[
          jax.ShapeDtypeStruct((N, D), jnp.float32),
          jax.ShapeDtypeStruct((N, D), jnp.float32),
      ],
  )(x, W1, degp)

  agg = _make_agg_kernel(ept)
  a1 = agg(h1p, packed, ew)

  h2p = pl.pallas_call(
      _tc2,
      out_shape=jax.ShapeDtypeStruct((N, D), jnp.float32),
  )(a1, h1p, dis, b1.reshape(1, D), W2)

  a2 = agg(h2p, packed, ew)

  out = pl.pallas_call(
      _tc3,
      out_shape=jax.ShapeDtypeStruct((N, D), jnp.float32),
  )(a2, h2p, dis, b2.reshape(1, D))

  return out


# speedup vs baseline: 17.6508x; 1.0085x over previous
import functools

import jax
import jax.numpy as jnp
from jax import lax
from jax.experimental import pallas as pl
from jax.experimental.pallas import tpu as pltpu
from jax.experimental.pallas import tpu_sc as plsc

N = 10000
NP = 10112
D = 128
NC = 2
NS = 16
NW = NC * NS
CH = 80
IDX_BITS = 14
ROWS_PER_TILE = NP // NS

_mesh = plsc.VectorSubcoreMesh(core_axis_name="c", subcore_axis_name="s")


def _zero_vmem(buf, rows, width):
  z = jnp.zeros((16,), jnp.float32)
  for i in range(rows):
    r = buf.at[i]
    for j in range(width // 16):
      r[pl.ds(j * 16, 16)] = z


def _zero_stripe(zbuf, zrows, shared, base_row):
  nfull = ROWS_PER_TILE // zrows
  for k in range(nfull):
    pltpu.sync_copy(zbuf, shared.at[pl.ds(base_row + k * zrows, zrows)])
  rem = ROWS_PER_TILE - nfull * zrows
  if rem:
    pltpu.sync_copy(zbuf.at[pl.ds(0, rem)],
                    shared.at[pl.ds(base_row + nfull * zrows, rem)])


def _copy_out_stripe(shared, out_core, base_row):
  for k in range(4):
    pltpu.sync_copy(shared.at[pl.ds(base_row + k * CH, CH)],
                    out_core.at[pl.ds(base_row + k * CH, CH)])
  r = ROWS_PER_TILE - 4 * CH
  pltpu.sync_copy(shared.at[pl.ds(base_row + 4 * CH, r)],
                  out_core.at[pl.ds(base_row + 4 * CH, r)])


def _make_deg_kernel(ept):

  @functools.partial(
      pl.kernel,
      out_type=jax.ShapeDtypeStruct((NW, NP), jnp.float32),
      mesh=_mesh,
      compiler_params=pltpu.CompilerParams(needs_layout_passes=False),
      scratch_types=[
          pltpu.VMEM((ept,), jnp.int32),
          pltpu.VMEM((ept,), jnp.float32),
          pltpu.VMEM((NP,), jnp.float32),
      ],
  )
  def _deg(pk_hbm, ew_hbm, out_hbm, pk_v, ew_v, tab_v):
    c = lax.axis_index("c")
    s = lax.axis_index("s")
    wid = c * NS + s
    z = jnp.zeros((16,), jnp.float32)

    def zbody(i, carry):
      tab_v[pl.ds(i * 16, 16)] = z
      return carry

    lax.fori_loop(0, NP // 16, zbody, 0)
    pltpu.sync_copy(pk_hbm.at[pl.ds(wid * ept, ept)], pk_v)
    pltpu.sync_copy(ew_hbm.at[pl.ds(wid * ept, ept)], ew_v)

    def body(i, carry):
      col = lax.shift_right_logical(pk_v[pl.ds(i * 16, 16)], IDX_BITS)
      plsc.addupdate_scatter(tab_v, [col], ew_v[pl.ds(i * 16, 16)])
      return carry

    lax.fori_loop(0, ept // 16, body, 0)
    pltpu.sync_copy(tab_v, out_hbm.at[wid])

  return _deg


def _make_agg_kernel(ept):
  nch = ept // CH
  assert ept % CH == 0 and nch % 2 == 0
  niter = nch // 2
  mask = (1 << IDX_BITS) - 1

  @functools.partial(
      pl.kernel,
      out_type=jax.ShapeDtypeStruct((NC, NP, D), jnp.float32),
      mesh=_mesh,
      scratch_types=[
          pltpu.VMEM((ept,), jnp.int32),
          pltpu.VMEM((ept,), jnp.float32),
          pltpu.VMEM((CH,), jnp.int32),
          pltpu.VMEM((CH,), jnp.int32),
          pltpu.VMEM((CH,), jnp.int32),
          pltpu.VMEM((CH,), jnp.int32),
          pltpu.VMEM((CH, D), jnp.float32),
          pltpu.VMEM((CH, D), jnp.float32),
          pltpu.SemaphoreType.DMA,
          pltpu.SemaphoreType.DMA,
          pltpu.VMEM_SHARED((NP, D), jnp.float32),
      ],
  )
  def _agg(hp_hbm, pk_hbm, ew_hbm, out_hbm, pk_v, ew_v, ridx_a, ridx_b,
           cidx_a, cidx_b, buf_a, buf_b, sem_a, sem_b, acc_sh):
    c = lax.axis_index("c")
    s = lax.axis_index("s")
    wid = c * NS + s
    _zero_vmem(buf_a, CH, D)
    base_row = s * ROWS_PER_TILE
    _zero_stripe(buf_a, CH, acc_sh, base_row)

    pltpu.sync_copy(pk_hbm.at[pl.ds(wid * ept, ept)], pk_v)
    pltpu.sync_copy(ew_hbm.at[pl.ds(wid * ept, ept)], ew_v)
    plsc.subcore_barrier()

    dn = lax.GatherDimensionNumbers(
        offset_dims=(), collapsed_slice_dims=(0,), start_index_map=(0,))

    def unpack(g, ridx, cidx):
      for t in range(CH // 16):
        v = pk_v[pl.ds(g * CH + t * 16, 16)]
        ridx[pl.ds(t * 16, 16)] = v & mask
        cidx[pl.ds(t * 16, 16)] = lax.shift_right_logical(v, IDX_BITS)

    def scale_scatter(g, buf, cidx):
      for t in range(CH // 16):
        ewg = ew_v[pl.ds(g * CH + t * 16, 16)]
        for l in range(16):
          w = lax.gather(ewg, jnp.full((16, 1), l, jnp.int32), dn,
                         slice_sizes=(1,),
                         mode=lax.GatherScatterMode.PROMISE_IN_BOUNDS)
          rv = buf.at[t * 16 + l]
          for j in range(D // 16):
            rv[pl.ds(j * 16, 16)] = rv[pl.ds(j * 16, 16)] * w
      pltpu.sync_copy(buf, acc_sh.at[cidx], add=True)

    unpack(0, ridx_a, cidx_a)
    pltpu.async_copy(hp_hbm.at[ridx_a], buf_a, sem_a)

    def body(k, carry):
      g = 2 * k
      unpack(g + 1, ridx_b, cidx_b)
      pltpu.async_copy(hp_hbm.at[ridx_b], buf_b, sem_b)
      pltpu.make_async_copy(hp_hbm.at[ridx_a], buf_a, sem_a).wait()
      scale_scatter(g, buf_a, cidx_a)

      @pl.when(k < niter - 1)
      def _():
        unpack(g + 2, ridx_a, cidx_a)
        pltpu.async_copy(hp_hbm.at[ridx_a], buf_a, sem_a)

      pltpu.make_async_copy(hp_hbm.at[ridx_b], buf_b, sem_b).wait()
      scale_scatter(g + 1, buf_b, cidx_b)
      return carry

    lax.fori_loop(0, niter, body, 0)
    plsc.subcore_barrier()
    _copy_out_stripe(acc_sh, out_hbm.at[c], base_row)

  return _agg


def _tc1(x_ref, w1_ref, degp_ref, h1p_ref, dis_ref):
  ones = jnp.ones((NW, D), jnp.float32)
  deg2d = lax.dot_general(degp_ref[...], ones, (((0,), (0,)), ((), ())),
                          preferred_element_type=jnp.float32)
  dis = lax.rsqrt(deg2d[:N] + 1.0)
  h = jnp.dot(x_ref[...], w1_ref[...], preferred_element_type=jnp.float32)
  h1p_ref[...] = h * dis
  dis_ref[...] = dis


def _tc2(ap_ref, hp_ref, dis_ref, b_ref, w2_ref, h2p_ref):
  dis = dis_ref[...]
  z = (ap_ref[0, :N] + ap_ref[1, :N] + hp_ref[...]) * dis + b_ref[...]
  y = jnp.where(z > 0, z, 0.01 * z)
  h2 = jnp.dot(y, w2_ref[...], preferred_element_type=jnp.float32)
  h2p_ref[...] = h2 * dis


def _tc3(ap_ref, hp_ref, dis_ref, b_ref, out_ref):
  out_ref[...] = (ap_ref[0, :N] + ap_ref[1, :N] + hp_ref[...]) \
      * dis_ref[...] + b_ref[...]


@jax.jit
def kernel(x, edge_index, edge_weight, W1, b1, W2, b2):
  e = edge_index.shape[1]
  ept = -(-e // (NW * 2 * CH)) * 2 * CH
  ep = NW * ept
  row = edge_index[0].astype(jnp.int32)
  col = edge_index[1].astype(jnp.int32)
  packed = jnp.pad(row | (col << IDX_BITS), (0, ep - e))
  ew = jnp.pad(edge_weight, (0, ep - e))

  degp = _make_deg_kernel(ept)(packed, ew)

  h1p, dis = pl.pallas_call(
      _tc1,
      out_shape=[
          jax.ShapeDtypeStruct((N, D), jnp.float32),
          jax.ShapeDtypeStruct((N, D), jnp.float32),
      ],
  )(x, W1, degp)

  agg = _make_agg_kernel(ept)
  a1 = agg(h1p, packed, ew)

  h2p = pl.pallas_call(
      _tc2,
      out_shape=jax.ShapeDtypeStruct((N, D), jnp.float32),
  )(a1, h1p, dis, b1.reshape(1, D), W2)

  a2 = agg(h2p, packed, ew)

  out = pl.pallas_call(
      _tc3,
      out_shape=jax.ShapeDtypeStruct((N, D), jnp.float32),
  )(a2, h2p, dis, b2.reshape(1, D))

  return out
